# Initial kernel scaffold; baseline (speedup 1.0000x reference)
import jax
import jax.numpy as jnp
from jax.experimental import pallas as pl


def kernel(efeat, nfeat, equery, src, dst, etype, rel_emb, gru_Wi, gru_Wh, gru_bi, gru_bh, pna_W, pna_b, lstm_Wx, lstm_Wh, lstm_b, ln_g, ln_b):
    raise NotImplementedError("write your pallas kernel here")


# speedup vs baseline: 1.4407x; 1.4407x over previous
import functools

import jax
import jax.numpy as jnp
from jax import lax
from jax.experimental import pallas as pl
from jax.experimental.pallas import tpu as pltpu
from jax.experimental.pallas import tpu_sc as plsc

_NC = 2
_NS = 16
_NW = _NC * _NS
_L = 16


def _sc_gather(table, idx):
    e_tot = idx.shape[0]
    d = table.shape[1]
    assert e_tot % _NW == 0
    rows_pw = e_tot // _NW
    G = 128
    nb_full = rows_pw // G
    rem = rows_pw - nb_full * G
    mesh = plsc.VectorSubcoreMesh(core_axis_name="c", subcore_axis_name="s", num_cores=_NC, num_subcores=_NS)

    scratch = [
        pltpu.VMEM((G,), jnp.int32),
        pltpu.VMEM((G, d), jnp.float32),
        pltpu.SemaphoreType.DMA,
    ]
    if rem:
        scratch += [
            pltpu.VMEM((rem,), jnp.int32),
            pltpu.VMEM((rem, d), jnp.float32),
        ]

    @functools.partial(
        pl.kernel,
        mesh=mesh,
        out_type=jax.ShapeDtypeStruct((e_tot, d), jnp.float32),
        scratch_types=scratch,
        compiler_params=pltpu.CompilerParams(needs_layout_passes=False),
    )
    def k(table_hbm, idx_hbm, out_hbm, idx_v, rows_v, sem, *rest):
        wid = lax.axis_index("s") * _NC + lax.axis_index("c")
        base = wid * rows_pw

        def body(b, _):
            off = base + b * G
            pltpu.sync_copy(idx_hbm.at[pl.ds(off, G)], idx_v)
            pltpu.async_copy(table_hbm.at[idx_v], rows_v, sem).wait()
            pltpu.sync_copy(rows_v, out_hbm.at[pl.ds(off, G)])
            return 0

        lax.fori_loop(0, nb_full, body, 0)
        if rem:
            idx_r, rows_r = rest
            off = base + nb_full * G
            pltpu.sync_copy(idx_hbm.at[pl.ds(off, rem)], idx_r)
            pltpu.async_copy(table_hbm.at[idx_r], rows_r, sem).wait()
            pltpu.sync_copy(rows_r, out_hbm.at[pl.ds(off, rem)])

    return k(table, idx)


def _seg_dims(n_nodes):
    NR = 3 * _NW
    RW = (-(-n_nodes // NR) + 7) // 8 * 8
    NP = NR * RW
    return NR, RW, NP


def _sc_segment(msg, dst, src, acc_init, deg_init, n_nodes):
    e_tot = dst.shape[0]
    d = msg.shape[1]
    NR, RW, NP = _seg_dims(n_nodes)
    CE = 3200
    assert e_tot % CE == 0
    n_chunks = e_tot // CE
    G = 48
    MB = (-(-CE // G)) * G + G

    mesh = plsc.VectorSubcoreMesh(core_axis_name="c", subcore_axis_name="s", num_cores=_NC, num_subcores=_NS)
    fvec = jax.ShapeDtypeStruct((NP, d), jnp.float32)
    f16v = jax.ShapeDtypeStruct((NP, _L), jnp.float32)

    @functools.partial(
        pl.kernel,
        mesh=mesh,
        out_type=(fvec, fvec, fvec, fvec, f16v, f16v),
        scratch_types=[
            pltpu.VMEM((RW, d), jnp.float32),
            pltpu.VMEM((RW, d), jnp.float32),
            pltpu.VMEM((RW, d), jnp.float32),
            pltpu.VMEM((RW, d), jnp.float32),
            pltpu.VMEM((RW, _L), jnp.float32),
            pltpu.VMEM((RW, _L), jnp.float32),
            pltpu.VMEM((CE,), jnp.int32),
            pltpu.VMEM((MB,), jnp.int32),
            pltpu.VMEM((MB,), jnp.int32),
            pltpu.VMEM((G, d), jnp.float32),
            pltpu.VMEM((G,), jnp.int32),
            pltpu.SemaphoreType.DMA,
        ],
        compiler_params=pltpu.CompilerParams(needs_layout_passes=False),
    )
    def k(msg_hbm, dst_hbm, src_hbm, ainit_hbm, dinit_hbm,
          sum_o, sq_o, mx_o, mn_o, deg_o, odeg_o,
          s_sum, s_sq, s_mx, s_mn, s_deg, s_odeg,
          chunk, midx, mrow, rowbuf, gidx, sem):
        wid = lax.axis_index("s") * _NC + lax.axis_index("c")
        zero16i = jnp.zeros((_L,), jnp.int32)
        iota16 = lax.broadcasted_iota(jnp.int32, (_L,), 0)
        e0 = jnp.where(iota16 == 0, 1.0, 0.0).astype(jnp.float32)

        def clr(i, _):
            midx[pl.ds(i * _L, _L)] = zero16i
            return 0
        lax.fori_loop(0, MB // _L, clr, 0)

        def do_range(j, _):
            r = wid * (NR // _NW) + j
            lo = r * RW

            pltpu.sync_copy(ainit_hbm.at[0], s_sum)
            pltpu.sync_copy(ainit_hbm.at[1], s_sq)
            pltpu.sync_copy(ainit_hbm.at[2], s_mx)
            pltpu.sync_copy(ainit_hbm.at[3], s_mn)
            pltpu.sync_copy(dinit_hbm, s_deg)
            pltpu.sync_copy(dinit_hbm, s_odeg)

            def do_chunk(ch, _):
                cbase = ch * CE

                pltpu.sync_copy(dst_hbm.at[pl.ds(cbase, CE)], chunk)

                def scan_g(g, cnt):
                    dv = chunk[pl.ds(g * _L, _L)]
                    rel = dv - lo
                    m = (rel >= 0) & (rel < RW)
                    eids = iota16 + (cbase + g * _L)
                    pos = plsc.cumsum(m.astype(jnp.int32)) + (cnt - 1)
                    plsc.store_scatter(midx, [pos], eids, mask=m)
                    plsc.store_scatter(mrow, [pos], rel, mask=m)
                    return cnt + jnp.sum(m.astype(jnp.int32))

                cnt = lax.fori_loop(0, CE // _L, scan_g, 0)
                plsc.store_scatter(midx, [cnt + iota16], zero16i)

                ngroup = (cnt + G - 1) // G

                def drain(gi, _):
                    gbase = gi * G
                    for t in range(G // _L):
                        gidx[pl.ds(t * _L, _L)] = midx[pl.ds(gbase + t * _L, _L)]
                    pltpu.async_copy(
                        msg_hbm.at[gidx], rowbuf, sem
                    ).wait()
                    ne = jnp.minimum(G, cnt - gbase)

                    def acc_one(i, _):
                        row = mrow[pl.ds(gbase + i, _L)][0]
                        for kk in range(d // _L):
                            sl = pl.ds(kk * _L, _L)
                            v = rowbuf[i, sl]
                            plsc.addupdate(s_sum.at[row, sl], v)
                            plsc.addupdate(s_sq.at[row, sl], v * v)
                            s_mx[row, sl] = jnp.maximum(s_mx[row, sl], v)
                            s_mn[row, sl] = jnp.minimum(s_mn[row, sl], v)
                        plsc.addupdate(s_deg.at[row, :], e0)
                        return 0

                    lax.fori_loop(0, ne, acc_one, 0)
                    return 0

                lax.fori_loop(0, ngroup, drain, 0)

                pltpu.sync_copy(src_hbm.at[pl.ds(cbase, CE)], chunk)

                def scan_s(g, cnt):
                    sv = chunk[pl.ds(g * _L, _L)]
                    rel = sv - lo
                    m = (rel >= 0) & (rel < RW)
                    pos = plsc.cumsum(m.astype(jnp.int32)) + (cnt - 1)
                    plsc.store_scatter(mrow, [pos], rel, mask=m)
                    return cnt + jnp.sum(m.astype(jnp.int32))

                cnt2 = lax.fori_loop(0, CE // _L, scan_s, 0)

                def acc_src(i, _):
                    row = mrow[pl.ds(i, _L)][0]
                    plsc.addupdate(s_odeg.at[row, :], e0)
                    return 0

                lax.fori_loop(0, cnt2, acc_src, 0)
                return 0

            lax.fori_loop(0, n_chunks, do_chunk, 0)

            pltpu.sync_copy(s_sum, sum_o.at[pl.ds(lo, RW)])
            pltpu.sync_copy(s_sq, sq_o.at[pl.ds(lo, RW)])
            pltpu.sync_copy(s_mx, mx_o.at[pl.ds(lo, RW)])
            pltpu.sync_copy(s_mn, mn_o.at[pl.ds(lo, RW)])
            pltpu.sync_copy(s_deg, deg_o.at[pl.ds(lo, RW)])
            pltpu.sync_copy(s_odeg, odeg_o.at[pl.ds(lo, RW)])
            return 0

        lax.fori_loop(0, NR // _NW, do_range, 0)

    return k(msg, dst, src, acc_init, deg_init)


def _gru_body(ef_ref, h_ref, et_ref, rel_ref, wi_ref, wh_ref, bi_ref, bh_ref,
              msg_ref):
    d = ef_ref.shape[1]
    et = et_ref[0, 0, :]
    oh = (et[:, None] == lax.broadcasted_iota(jnp.int32, (1, rel_ref.shape[0]), 1)
          ).astype(jnp.float32)
    x = ef_ref[...] + jnp.dot(oh, rel_ref[...],
                              preferred_element_type=jnp.float32)
    h = h_ref[...]
    gi = jnp.dot(x, wi_ref[...], preferred_element_type=jnp.float32) + bi_ref[...]
    gh = jnp.dot(h, wh_ref[...], preferred_element_type=jnp.float32) + bh_ref[...]
    r = jax.nn.sigmoid(gi[:, 0:d] + gh[:, 0:d])
    z = jax.nn.sigmoid(gi[:, d:2 * d] + gh[:, d:2 * d])
    ng = jnp.tanh(gi[:, 2 * d:] + r * gh[:, 2 * d:])
    msg_ref[...] = (1.0 - z) * ng + z * h


def _tc_gru(efeat, hsrc, etype3, rel_emb, gru_Wi, gru_Wh, bi2, bh2, be):
    e_tot, d = efeat.shape
    nb = e_tot // be
    r = rel_emb.shape[0]
    grid = (nb,)
    return pl.pallas_call(
        _gru_body,
        grid=grid,
        in_specs=[
            pl.BlockSpec((be, d), lambda i: (i, 0)),
            pl.BlockSpec((be, d), lambda i: (i, 0)),
            pl.BlockSpec((1, 1, be), lambda i: (i, 0, 0)),
            pl.BlockSpec((r, d), lambda i: (0, 0)),
            pl.BlockSpec((d, 3 * d), lambda i: (0, 0)),
            pl.BlockSpec((d, 3 * d), lambda i: (0, 0)),
            pl.BlockSpec((1, 3 * d), lambda i: (0, 0)),
            pl.BlockSpec((1, 3 * d), lambda i: (0, 0)),
        ],
        out_specs=pl.BlockSpec((be, d), lambda i: (i, 0)),
        out_shape=jax.ShapeDtypeStruct((e_tot, d), jnp.float32),
    )(efeat, hsrc, etype3, rel_emb, gru_Wi, gru_Wh, bi2, bh2)


def _avgd_body(od_ref, out_ref, *, n_nodes):
    rows, cols = od_ref.shape
    ridx = lax.broadcasted_iota(jnp.int32, (rows, cols), 0)
    cidx = lax.broadcasted_iota(jnp.int32, (rows, cols), 1)
    flat = ridx * cols + cidx
    m = ((cidx % _L) == 0) & (flat < n_nodes * _L)
    v = jnp.where(m, od_ref[...], 0.0)
    total = jnp.sum(jnp.log(v + 1.0) * m.astype(jnp.float32))
    out_ref[...] = jnp.full((8, 128), total / n_nodes, jnp.float32)


def _tc_avgd(odeg_flat, n_nodes):
    rows, cols = odeg_flat.shape
    return pl.pallas_call(
        functools.partial(_avgd_body, n_nodes=n_nodes),
        grid=(1,),
        in_specs=[pl.BlockSpec((rows, cols), lambda i: (0, 0))],
        out_specs=pl.BlockSpec((8, 128), lambda i: (0, 0)),
        out_shape=jax.ShapeDtypeStruct((8, 128), jnp.float32),
    )(odeg_flat)


def _pna_body(sum_ref, sq_ref, mx_ref, mn_ref, deg_ref, avg_ref, w_ref, b_ref,
              g_ref, bb_ref, raw_ref, ln_ref):
    deg = deg_ref[...]
    avg_d = avg_ref[0, 0]
    safe = jnp.maximum(deg, 1.0)
    mean = sum_ref[...] / safe
    var = jnp.maximum(sq_ref[...] / safe - mean * mean, 0.0)
    std = jnp.sqrt(var + 1e-5)
    has = deg > 0.0
    mx = jnp.where(has, mx_ref[...], 0.0)
    mn = jnp.where(has, mn_ref[...], 0.0)
    logd = jnp.log(deg[:, 0:1] + 1.0)
    amp = logd / avg_d
    att = jnp.where(has[:, 0:1], avg_d / jnp.maximum(logd, 1e-5), 0.0)
    aggs = jnp.concatenate([mean, mx, mn, std], axis=-1)
    comb = jnp.concatenate([aggs, aggs * amp, aggs * att], axis=-1)
    nf = jnp.dot(comb, w_ref[...], preferred_element_type=jnp.float32) + b_ref[...]
    raw_ref[...] = nf
    mu = jnp.mean(nf, axis=-1, keepdims=True)
    vv = jnp.mean((nf - mu) * (nf - mu), axis=-1, keepdims=True)
    ln_ref[...] = (nf - mu) / jnp.sqrt(vv + 1e-5) * g_ref[...] + bb_ref[...]


def _tc_pna(sum_a, sq_a, mx_a, mn_a, degb, avg, pna_W, pb2, g2, b2, bn):
    np_, d = sum_a.shape
    nb = np_ // bn
    blk = lambda i: (i, 0)
    whole = lambda i: (0, 0)
    shp = jax.ShapeDtypeStruct((np_, d), jnp.float32)
    return pl.pallas_call(
        _pna_body,
        grid=(nb,),
        in_specs=[
            pl.BlockSpec((bn, d), blk),
            pl.BlockSpec((bn, d), blk),
            pl.BlockSpec((bn, d), blk),
            pl.BlockSpec((bn, d), blk),
            pl.BlockSpec((bn, d), blk),
            pl.BlockSpec((8, 128), whole),
            pl.BlockSpec((12 * d, d), whole),
            pl.BlockSpec((1, d), whole),
            pl.BlockSpec((1, d), whole),
            pl.BlockSpec((1, d), whole),
        ],
        out_specs=[pl.BlockSpec((bn, d), blk), pl.BlockSpec((bn, d), blk)],
        out_shape=[shp, shp],
    )(sum_a, sq_a, mx_a, mn_a, degb, avg, pna_W, pb2, g2, b2)


def _lstm_body(p_ref, ef_ref, eq_ref, wx_ref, wh_ref, b_ref, g_ref, bb_ref,
               h_ref, c_ref):
    d = ef_ref.shape[1]
    gates = (jnp.dot(p_ref[...], wx_ref[...], preferred_element_type=jnp.float32)
             + jnp.dot(ef_ref[...], wh_ref[...], preferred_element_type=jnp.float32)
             + b_ref[...])
    ig = jax.nn.sigmoid(gates[:, 0:d])
    fg = jax.nn.sigmoid(gates[:, d:2 * d])
    gg = jnp.tanh(gates[:, 2 * d:3 * d])
    og = jax.nn.sigmoid(gates[:, 3 * d:])
    new_c = fg * eq_ref[...] + ig * gg
    new_h = og * jnp.tanh(new_c)

    def ln(x):
        mu = jnp.mean(x, axis=-1, keepdims=True)
        vv = jnp.mean((x - mu) * (x - mu), axis=-1, keepdims=True)
        return (x - mu) / jnp.sqrt(vv + 1e-5) * g_ref[...] + bb_ref[...]

    h_ref[...] = ln(new_h)
    c_ref[...] = ln(new_c)


def _tc_lstm(pdst, efeat, equery, wx, wh, b2, g2, bb2, be):
    e_tot, d = efeat.shape
    nb = e_tot // be
    blk = lambda i: (i, 0)
    whole = lambda i: (0, 0)
    shp = jax.ShapeDtypeStruct((e_tot, d), jnp.float32)
    return pl.pallas_call(
        _lstm_body,
        grid=(nb,),
        in_specs=[
            pl.BlockSpec((be, d), blk),
            pl.BlockSpec((be, d), blk),
            pl.BlockSpec((be, d), blk),
            pl.BlockSpec((d, 4 * d), whole),
            pl.BlockSpec((d, 4 * d), whole),
            pl.BlockSpec((1, 4 * d), whole),
            pl.BlockSpec((1, d), whole),
            pl.BlockSpec((1, d), whole),
        ],
        out_specs=[pl.BlockSpec((be, d), blk), pl.BlockSpec((be, d), blk)],
        out_shape=[shp, shp],
    )(pdst, efeat, equery, wx, wh, b2, g2, bb2)


def kernel(efeat, nfeat, equery, src, dst, etype, rel_emb, gru_Wi, gru_Wh,
           gru_bi, gru_bh, pna_W, pna_b, lstm_Wx, lstm_Wh, lstm_b, ln_g, ln_b):
    e_tot, d = efeat.shape
    n = nfeat.shape[0]
    be = 2000
    NR, RW, NP = _seg_dims(n)

    src32 = src.astype(jnp.int32)
    dst32 = dst.astype(jnp.int32)
    etype3 = etype.astype(jnp.int32).reshape(e_tot // be, 1, be)

    big = jnp.float32(3.0e38)
    acc_init = jnp.stack([
        jnp.zeros((RW, d), jnp.float32),
        jnp.zeros((RW, d), jnp.float32),
        jnp.full((RW, d), -big, jnp.float32),
        jnp.full((RW, d), big, jnp.float32),
    ])
    deg_init = jnp.zeros((RW, _L), jnp.float32)

    hsrc = _sc_gather(nfeat, src32)
    msg = _tc_gru(efeat, hsrc, etype3, rel_emb, gru_Wi, gru_Wh,
                  gru_bi.reshape(1, -1), gru_bh.reshape(1, -1), be)
    sum_a, sq_a, mx_a, mn_a, deg16, odeg16 = _sc_segment(
        msg, dst32, src32, acc_init, deg_init, n)

    avg = _tc_avgd(odeg16.reshape(-1, 128), n)
    degb = jnp.broadcast_to(deg16[:, :1], (NP, d))

    nf_raw, nf_ln = _tc_pna(sum_a, sq_a, mx_a, mn_a, degb, avg, pna_W,
                            pna_b.reshape(1, -1), ln_g.reshape(1, -1),
                            ln_b.reshape(1, -1), NP // 8)

    pdst = _sc_gather(nf_raw, dst32)
    h_ln, c_ln = _tc_lstm(pdst, efeat, equery, lstm_Wx, lstm_Wh,
                          lstm_b.reshape(1, -1), ln_g.reshape(1, -1),
                          ln_b.reshape(1, -1), be)
    return h_ln, nf_ln[:n], c_ln


# speedup vs baseline: 2.0199x; 1.4020x over previous
import functools

import jax
import jax.numpy as jnp
from jax import lax
from jax.experimental import pallas as pl
from jax.experimental.pallas import tpu as pltpu
from jax.experimental.pallas import tpu_sc as plsc

_NC = 2
_NS = 16
_NW = _NC * _NS
_L = 16


def _sc_gather(table, idx):
    e_tot = idx.shape[0]
    d = table.shape[1]
    assert e_tot % _NW == 0
    rows_pw = e_tot // _NW
    G = 128
    nb_full = rows_pw // G
    rem = rows_pw - nb_full * G
    nb2 = nb_full // 2
    nb_odd = nb_full - nb2 * 2
    mesh = plsc.VectorSubcoreMesh(core_axis_name="c", subcore_axis_name="s", num_cores=_NC, num_subcores=_NS)

    scratch = [
        pltpu.VMEM((G,), jnp.int32),
        pltpu.VMEM((G,), jnp.int32),
        pltpu.VMEM((G, d), jnp.float32),
        pltpu.VMEM((G, d), jnp.float32),
        pltpu.SemaphoreType.DMA,
        pltpu.SemaphoreType.DMA,
        pltpu.SemaphoreType.DMA,
        pltpu.SemaphoreType.DMA,
    ]
    if rem:
        scratch += [
            pltpu.VMEM((rem,), jnp.int32),
            pltpu.VMEM((rem, d), jnp.float32),
        ]

    @functools.partial(
        pl.kernel,
        mesh=mesh,
        out_type=jax.ShapeDtypeStruct((e_tot, d), jnp.float32),
        scratch_types=scratch,
        compiler_params=pltpu.CompilerParams(needs_layout_passes=False),
    )
    def k(table_hbm, idx_hbm, out_hbm, iv0, iv1, rv0, rv1,
          gs0, gs1, ws0, ws1, *rest):
        wid = lax.axis_index("s") * _NC + lax.axis_index("c")
        base = wid * rows_pw
        ivs, rvs = (iv0, iv1), (rv0, rv1)
        gss, wss = (gs0, gs1), (ws0, ws1)

        def start_gather(b, p):
            pltpu.sync_copy(idx_hbm.at[pl.ds(base + b * G, G)], ivs[p])
            pltpu.async_copy(table_hbm.at[ivs[p]], rvs[p], gss[p])

        def wait_gather(p):
            pltpu.make_async_copy(table_hbm.at[ivs[p]], rvs[p],
                                  gss[p]).wait()

        def start_write(b, p):
            pltpu.async_copy(rvs[p], out_hbm.at[pl.ds(base + b * G, G)],
                             wss[p])

        def wait_write(p):
            pltpu.make_async_copy(rvs[p], out_hbm.at[pl.ds(base, G)],
                                  wss[p]).wait()

        if nb_full:
            start_gather(0, 0)
        if nb_full > 1:
            start_gather(1, 1)

        def pair(bb, _):
            for p in (0, 1):
                b = bb * 2 + p
                wait_gather(p)
                start_write(b, p)
                bn = b + 2

                def nxt():
                    wait_write(p)
                    start_gather(bn, p)
                pl.when(bn < nb_full)(nxt)
            return 0

        lax.fori_loop(0, nb2, pair, 0)
        if nb_odd:
            b = nb2 * 2
            wait_gather(0)
            start_write(b, 0)
        if rem:
            idx_r, rows_r = rest
            off = base + nb_full * G
            pltpu.sync_copy(idx_hbm.at[pl.ds(off, rem)], idx_r)
            pltpu.async_copy(table_hbm.at[idx_r], rows_r, gs0).wait()
            pltpu.sync_copy(rows_r, out_hbm.at[pl.ds(off, rem)])
        if nb_full:
            wait_write(0)
        if nb_full > 1:
            wait_write(1)

    return k(table, idx)


def _seg_dims(n_nodes):
    NR = 3 * _NW
    RW = (-(-n_nodes // NR) + 7) // 8 * 8
    NP = NR * RW
    return NR, RW, NP


def _sc_segment(msg, dst, src, acc_init, deg_init, n_nodes):
    e_tot = dst.shape[0]
    d = msg.shape[1]
    NR, RW, NP = _seg_dims(n_nodes)
    RPW = NR // _NW
    assert RW <= 128
    W3 = RPW * RW
    CE = 2000
    assert e_tot % CE == 0 and (e_tot // CE) % 2 == 0
    n_chunks = e_tot // CE
    G = 32
    MB = (-(-CE // G)) * G + G

    mesh = plsc.VectorSubcoreMesh(core_axis_name="c", subcore_axis_name="s", num_cores=_NC, num_subcores=_NS)
    fvec = jax.ShapeDtypeStruct((NP, d), jnp.float32)
    f16v = jax.ShapeDtypeStruct((NP, _L), jnp.float32)

    @functools.partial(
        pl.kernel,
        mesh=mesh,
        out_type=(fvec, fvec, fvec, fvec, f16v, f16v),
        scratch_types=[
            pltpu.VMEM((RW, d), jnp.float32),
            pltpu.VMEM((RW, d), jnp.float32),
            pltpu.VMEM((RW, d), jnp.float32),
            pltpu.VMEM((RW, d), jnp.float32),
            pltpu.VMEM((RW, _L), jnp.float32),
            pltpu.VMEM((W3, _L), jnp.float32),
            pltpu.VMEM((CE,), jnp.int32),
            pltpu.VMEM((CE,), jnp.int32),
            pltpu.VMEM((MB,), jnp.int32),
            pltpu.VMEM((MB,), jnp.int32),
            pltpu.VMEM((G, d), jnp.float32),
            pltpu.VMEM((G,), jnp.int32),
            pltpu.SemaphoreType.DMA,
            pltpu.SemaphoreType.DMA,
        ],
        compiler_params=pltpu.CompilerParams(needs_layout_passes=False),
    )
    def k(msg_hbm, dst_hbm, src_hbm, ainit_hbm, dinit_hbm,
          sum_o, sq_o, mx_o, mn_o, deg_o, odeg_o,
          s_sum, s_sq, s_mx, s_mn, s_deg, s_odeg,
          chunkA, chunkB, midxA, midxB, rowbuf, gidx, gsem, csem):
        wid = lax.axis_index("s") * _NC + lax.axis_index("c")
        zero16i = jnp.zeros((_L,), jnp.int32)
        iota16 = lax.broadcasted_iota(jnp.int32, (_L,), 0)
        e0 = jnp.where(iota16 == 0, 1.0, 0.0).astype(jnp.float32)
        chunks = (chunkA, chunkB)
        midxs = (midxA, midxB)

        def clr(i, _):
            midxA[pl.ds(i * _L, _L)] = zero16i
            midxB[pl.ds(i * _L, _L)] = zero16i
            return 0
        lax.fori_loop(0, MB // _L, clr, 0)

        def fetch_chunk(src_ref, i, p):
            pltpu.async_copy(src_ref.at[pl.ds(i * CE, CE)], chunks[p], csem)

        def wait_chunk(p):
            pltpu.make_async_copy(dst_hbm.at[pl.ds(0, CE)], chunks[p],
                                  csem).wait()

        lo0 = wid * W3
        for k3 in range(RPW):
            pltpu.sync_copy(dinit_hbm, s_odeg.at[pl.ds(k3 * RW, RW)])

        def od_scan(i, p):
            def scan_s(g, cnt):
                sv = chunks[p][pl.ds(g * _L, _L)]
                rel = sv - lo0
                m = rel.astype(jnp.uint32) < jnp.uint32(W3)
                ng = plsc.all_reduce_population_count(m)[0]

                def st():
                    pos = plsc.cumsum(m.astype(jnp.int32)) + (cnt - 1)
                    plsc.store_scatter(midxs[p], [pos], rel, mask=m)
                pl.when(ng > 0)(st)
                return cnt + ng

            cnt2 = lax.fori_loop(0, CE // _L, scan_s, 0)

            def acc_src(i2, _):
                row = midxs[p][pl.ds(i2, _L)][0]
                plsc.addupdate(s_odeg.at[row, :], e0)
                return 0

            lax.fori_loop(0, cnt2, acc_src, 0)
            return 0

        fetch_chunk(src_hbm, 0, 0)
        wait_chunk(0)

        def od_pair(t, _):
            for p in (0, 1):
                i = t * 2 + p
                q = 1 - p

                def pre():
                    fetch_chunk(src_hbm, i + 1, q)
                pl.when(i + 1 < n_chunks)(pre)
                od_scan(i, p)

                def wt():
                    wait_chunk(q)
                pl.when(i + 1 < n_chunks)(wt)
            return 0

        lax.fori_loop(0, n_chunks // 2, od_pair, 0)
        pltpu.sync_copy(s_odeg, odeg_o.at[pl.ds(lo0, W3)])

        def do_range(j, _):
            r = wid * RPW + j
            lo = r * RW

            pltpu.sync_copy(ainit_hbm.at[0], s_sum)
            pltpu.sync_copy(ainit_hbm.at[1], s_sq)
            pltpu.sync_copy(ainit_hbm.at[2], s_mx)
            pltpu.sync_copy(ainit_hbm.at[3], s_mn)
            pltpu.sync_copy(dinit_hbm, s_deg)

            def scan(i, p):
                cbase = i * CE

                def scan_g(g, cnt):
                    dv = chunks[p][pl.ds(g * _L, _L)]
                    rel = dv - lo
                    m = rel.astype(jnp.uint32) < jnp.uint32(RW)
                    ng = plsc.all_reduce_population_count(m)[0]

                    def st():
                        packed = ((iota16 + (cbase + g * _L)) << 7) | rel
                        pos = plsc.cumsum(m.astype(jnp.int32)) + (cnt - 1)
                        plsc.store_scatter(midxs[p], [pos], packed, mask=m)
                    pl.when(ng > 0)(st)
                    return cnt + ng

                return lax.fori_loop(0, CE // _L, scan_g, 0)

            def stage_and_fire(p, gbase):
                for t in range(G // _L):
                    gidx[pl.ds(t * _L, _L)] = (
                        midxs[p][pl.ds(gbase + t * _L, _L)] >> 7)
                pltpu.async_copy(msg_hbm.at[gidx], rowbuf, gsem)

            def wait_gather():
                pltpu.make_async_copy(msg_hbm.at[gidx], rowbuf, gsem).wait()

            def accumulate(p, gbase, cnt):
                ne = jnp.minimum(G, cnt - gbase)

                def acc_one(i2, _):
                    row = midxs[p][pl.ds(gbase + i2, _L)][0] & 127
                    for kk in range(d // _L):
                        sl = pl.ds(kk * _L, _L)
                        v = rowbuf[i2, sl]
                        plsc.addupdate(s_sum.at[row, sl], v)
                        plsc.addupdate(s_sq.at[row, sl], v * v)
                        s_mx[row, sl] = jnp.maximum(s_mx[row, sl], v)
                        s_mn[row, sl] = jnp.minimum(s_mn[row, sl], v)
                    plsc.addupdate(s_deg.at[row, :], e0)
                    return 0

                lax.fori_loop(0, ne, acc_one, 0)

            def step(i, p, cnt_p):
                q = 1 - p
                ngrp = (cnt_p + G - 1) // G
                pl.when(ngrp > 0)(lambda: stage_and_fire(p, 0))
                pl.when(i + 1 < n_chunks)(
                    lambda: fetch_chunk(dst_hbm, i + 1, q))

                def drain0():
                    wait_gather()
                    accumulate(p, 0, cnt_p)

                    def more(gi, _):
                        stage_and_fire(p, gi * G)
                        wait_gather()
                        accumulate(p, gi * G, cnt_p)
                        return 0

                    lax.fori_loop(1, ngrp, more, 0)
                pl.when(ngrp > 0)(drain0)

                def nxt():
                    wait_chunk(q)
                    return scan(i + 1, q)
                return lax.cond(i + 1 < n_chunks, nxt, lambda: 0)

            fetch_chunk(dst_hbm, 0, 0)
            wait_chunk(0)
            c0 = scan(0, 0)

            def pair(t, cnt):
                cnt = step(t * 2, 0, cnt)
                cnt = step(t * 2 + 1, 1, cnt)
                return cnt

            lax.fori_loop(0, n_chunks // 2, pair, c0)

            pltpu.sync_copy(s_sum, sum_o.at[pl.ds(lo, RW)])
            pltpu.sync_copy(s_sq, sq_o.at[pl.ds(lo, RW)])
            pltpu.sync_copy(s_mx, mx_o.at[pl.ds(lo, RW)])
            pltpu.sync_copy(s_mn, mn_o.at[pl.ds(lo, RW)])
            pltpu.sync_copy(s_deg, deg_o.at[pl.ds(lo, RW)])
            return 0

        lax.fori_loop(0, RPW, do_range, 0)

    return k(msg, dst, src, acc_init, deg_init)


def _gru_body(ef_ref, h_ref, et_ref, rel_ref, wi_ref, wh_ref, bi_ref, bh_ref,
              msg_ref):
    d = ef_ref.shape[1]
    et = et_ref[0, 0, :]
    oh = (et[:, None] == lax.broadcasted_iota(jnp.int32, (1, rel_ref.shape[0]), 1)
          ).astype(jnp.float32)
    x = ef_ref[...] + jnp.dot(oh, rel_ref[...],
                              preferred_element_type=jnp.float32)
    h = h_ref[...]
    gi = jnp.dot(x, wi_ref[...], preferred_element_type=jnp.float32) + bi_ref[...]
    gh = jnp.dot(h, wh_ref[...], preferred_element_type=jnp.float32) + bh_ref[...]
    r = jax.nn.sigmoid(gi[:, 0:d] + gh[:, 0:d])
    z = jax.nn.sigmoid(gi[:, d:2 * d] + gh[:, d:2 * d])
    ng = jnp.tanh(gi[:, 2 * d:] + r * gh[:, 2 * d:])
    msg_ref[...] = (1.0 - z) * ng + z * h


def _tc_gru(efeat, hsrc, etype3, rel_emb, gru_Wi, gru_Wh, bi2, bh2, be):
    e_tot, d = efeat.shape
    nb = e_tot // be
    r = rel_emb.shape[0]
    grid = (nb,)
    return pl.pallas_call(
        _gru_body,
        grid=grid,
        in_specs=[
            pl.BlockSpec((be, d), lambda i: (i, 0)),
            pl.BlockSpec((be, d), lambda i: (i, 0)),
            pl.BlockSpec((1, 1, be), lambda i: (i, 0, 0)),
            pl.BlockSpec((r, d), lambda i: (0, 0)),
            pl.BlockSpec((d, 3 * d), lambda i: (0, 0)),
            pl.BlockSpec((d, 3 * d), lambda i: (0, 0)),
            pl.BlockSpec((1, 3 * d), lambda i: (0, 0)),
            pl.BlockSpec((1, 3 * d), lambda i: (0, 0)),
        ],
        out_specs=pl.BlockSpec((be, d), lambda i: (i, 0)),
        out_shape=jax.ShapeDtypeStruct((e_tot, d), jnp.float32),
    )(efeat, hsrc, etype3, rel_emb, gru_Wi, gru_Wh, bi2, bh2)


def _avgd_body(od_ref, out_ref, *, n_nodes):
    rows, cols = od_ref.shape
    ridx = lax.broadcasted_iota(jnp.int32, (rows, cols), 0)
    cidx = lax.broadcasted_iota(jnp.int32, (rows, cols), 1)
    flat = ridx * cols + cidx
    m = ((cidx % _L) == 0) & (flat < n_nodes * _L)
    v = jnp.where(m, od_ref[...], 0.0)
    total = jnp.sum(jnp.log(v + 1.0) * m.astype(jnp.float32))
    out_ref[...] = jnp.full((8, 128), total / n_nodes, jnp.float32)


def _tc_avgd(odeg_flat, n_nodes):
    rows, cols = odeg_flat.shape
    return pl.pallas_call(
        functools.partial(_avgd_body, n_nodes=n_nodes),
        grid=(1,),
        in_specs=[pl.BlockSpec((rows, cols), lambda i: (0, 0))],
        out_specs=pl.BlockSpec((8, 128), lambda i: (0, 0)),
        out_shape=jax.ShapeDtypeStruct((8, 128), jnp.float32),
    )(odeg_flat)


def _pna_body(sum_ref, sq_ref, mx_ref, mn_ref, deg_ref, avg_ref, w_ref, b_ref,
              g_ref, bb_ref, raw_ref, ln_ref):
    deg = deg_ref[...]
    avg_d = avg_ref[0, 0]
    safe = jnp.maximum(deg, 1.0)
    mean = sum_ref[...] / safe
    var = jnp.maximum(sq_ref[...] / safe - mean * mean, 0.0)
    std = jnp.sqrt(var + 1e-5)
    has = deg > 0.0
    mx = jnp.where(has, mx_ref[...], 0.0)
    mn = jnp.where(has, mn_ref[...], 0.0)
    logd = jnp.log(deg[:, 0:1] + 1.0)
    amp = logd / avg_d
    att = jnp.where(has[:, 0:1], avg_d / jnp.maximum(logd, 1e-5), 0.0)
    aggs = jnp.concatenate([mean, mx, mn, std], axis=-1)
    comb = jnp.concatenate([aggs, aggs * amp, aggs * att], axis=-1)
    nf = jnp.dot(comb, w_ref[...], preferred_element_type=jnp.float32) + b_ref[...]
    raw_ref[...] = nf
    mu = jnp.mean(nf, axis=-1, keepdims=True)
    vv = jnp.mean((nf - mu) * (nf - mu), axis=-1, keepdims=True)
    ln_ref[...] = (nf - mu) / jnp.sqrt(vv + 1e-5) * g_ref[...] + bb_ref[...]


def _tc_pna(sum_a, sq_a, mx_a, mn_a, degb, avg, pna_W, pb2, g2, b2, bn):
    np_, d = sum_a.shape
    nb = np_ // bn
    blk = lambda i: (i, 0)
    whole = lambda i: (0, 0)
    shp = jax.ShapeDtypeStruct((np_, d), jnp.float32)
    return pl.pallas_call(
        _pna_body,
        grid=(nb,),
        in_specs=[
            pl.BlockSpec((bn, d), blk),
            pl.BlockSpec((bn, d), blk),
            pl.BlockSpec((bn, d), blk),
            pl.BlockSpec((bn, d), blk),
            pl.BlockSpec((bn, d), blk),
            pl.BlockSpec((8, 128), whole),
            pl.BlockSpec((12 * d, d), whole),
            pl.BlockSpec((1, d), whole),
            pl.BlockSpec((1, d), whole),
            pl.BlockSpec((1, d), whole),
        ],
        out_specs=[pl.BlockSpec((bn, d), blk), pl.BlockSpec((bn, d), blk)],
        out_shape=[shp, shp],
    )(sum_a, sq_a, mx_a, mn_a, degb, avg, pna_W, pb2, g2, b2)


def _lstm_body(p_ref, ef_ref, eq_ref, wx_ref, wh_ref, b_ref, g_ref, bb_ref,
               h_ref, c_ref):
    d = ef_ref.shape[1]
    gates = (jnp.dot(p_ref[...], wx_ref[...], preferred_element_type=jnp.float32)
             + jnp.dot(ef_ref[...], wh_ref[...], preferred_element_type=jnp.float32)
             + b_ref[...])
    ig = jax.nn.sigmoid(gates[:, 0:d])
    fg = jax.nn.sigmoid(gates[:, d:2 * d])
    gg = jnp.tanh(gates[:, 2 * d:3 * d])
    og = jax.nn.sigmoid(gates[:, 3 * d:])
    new_c = fg * eq_ref[...] + ig * gg
    new_h = og * jnp.tanh(new_c)

    def ln(x):
        mu = jnp.mean(x, axis=-1, keepdims=True)
        vv = jnp.mean((x - mu) * (x - mu), axis=-1, keepdims=True)
        return (x - mu) / jnp.sqrt(vv + 1e-5) * g_ref[...] + bb_ref[...]

    h_ref[...] = ln(new_h)
    c_ref[...] = ln(new_c)


def _tc_lstm(pdst, efeat, equery, wx, wh, b2, g2, bb2, be):
    e_tot, d = efeat.shape
    nb = e_tot // be
    blk = lambda i: (i, 0)
    whole = lambda i: (0, 0)
    shp = jax.ShapeDtypeStruct((e_tot, d), jnp.float32)
    return pl.pallas_call(
        _lstm_body,
        grid=(nb,),
        in_specs=[
            pl.BlockSpec((be, d), blk),
            pl.BlockSpec((be, d), blk),
            pl.BlockSpec((be, d), blk),
            pl.BlockSpec((d, 4 * d), whole),
            pl.BlockSpec((d, 4 * d), whole),
            pl.BlockSpec((1, 4 * d), whole),
            pl.BlockSpec((1, d), whole),
            pl.BlockSpec((1, d), whole),
        ],
        out_specs=[pl.BlockSpec((be, d), blk), pl.BlockSpec((be, d), blk)],
        out_shape=[shp, shp],
    )(pdst, efeat, equery, wx, wh, b2, g2, bb2)


def kernel(efeat, nfeat, equery, src, dst, etype, rel_emb, gru_Wi, gru_Wh,
           gru_bi, gru_bh, pna_W, pna_b, lstm_Wx, lstm_Wh, lstm_b, ln_g, ln_b):
    e_tot, d = efeat.shape
    n = nfeat.shape[0]
    be = 2000
    NR, RW, NP = _seg_dims(n)

    src32 = src.astype(jnp.int32)
    dst32 = dst.astype(jnp.int32)
    etype3 = etype.astype(jnp.int32).reshape(e_tot // be, 1, be)

    big = jnp.float32(3.0e38)
    acc_init = jnp.stack([
        jnp.zeros((RW, d), jnp.float32),
        jnp.zeros((RW, d), jnp.float32),
        jnp.full((RW, d), -big, jnp.float32),
        jnp.full((RW, d), big, jnp.float32),
    ])
    deg_init = jnp.zeros((RW, _L), jnp.float32)

    hsrc = _sc_gather(nfeat, src32)
    msg = _tc_gru(efeat, hsrc, etype3, rel_emb, gru_Wi, gru_Wh,
                  gru_bi.reshape(1, -1), gru_bh.reshape(1, -1), be)
    sum_a, sq_a, mx_a, mn_a, deg16, odeg16 = _sc_segment(
        msg, dst32, src32, acc_init, deg_init, n)

    avg = _tc_avgd(odeg16.reshape(-1, 128), n)
    degb = jnp.broadcast_to(deg16[:, :1], (NP, d))

    nf_raw, nf_ln = _tc_pna(sum_a, sq_a, mx_a, mn_a, degb, avg, pna_W,
                            pna_b.reshape(1, -1), ln_g.reshape(1, -1),
                            ln_b.reshape(1, -1), NP // 8)

    pdst = _sc_gather(nf_raw, dst32)
    h_ln, c_ln = _tc_lstm(pdst, efeat, equery, lstm_Wx, lstm_Wh,
                          lstm_b.reshape(1, -1), ln_g.reshape(1, -1),
                          ln_b.reshape(1, -1), be)
    return h_ln, nf_ln[:n], c_ln
